# baseline (speedup 1.0000x reference)
import functools

import jax
import jax.numpy as jnp
from jax import lax
from jax.experimental import pallas as pl
from jax.experimental.pallas import tpu as pltpu
from jax.experimental.pallas import tpu_sc as plsc

NC = 2
NS = 16
CH = 125


def _mesh():
    return plsc.VectorSubcoreMesh(core_axis_name="c", subcore_axis_name="s",
                                  num_cores=NC, num_subcores=NS)


def _node_slab(s):
    return s * 624


def _make_deg(n, e):
    rows = e // CH
    rpt = rows // (NC * NS)

    @functools.partial(
        pl.kernel,
        out_type=jax.ShapeDtypeStruct((NC, n), jnp.float32),
        mesh=_mesh(),
        scratch_types=[
            pltpu.VMEM((rpt, CH), jnp.int32),
            pltpu.VMEM((128,), jnp.float32),
            pltpu.VMEM((n,), jnp.float32),
            pltpu.VMEM_SHARED((n,), jnp.float32),
        ],
    )
    def deg_kernel(dst_hbm, deg_hbm, idx_v, ones_v, zero_v, deg_sh):
        c = lax.axis_index("c")
        s = lax.axis_index("s")
        w = c * NS + s

        for j in range(8):
            ones_v[pl.ds(j * 16, 16)] = jnp.ones((16,), jnp.float32)

        @pl.when(s == 0)
        def _():
            def zero_body(i, _):
                zero_v[pl.ds(i * 16, 16)] = jnp.zeros((16,), jnp.float32)
                return 0

            lax.fori_loop(0, n // 16, zero_body, 0)
            pltpu.sync_copy(zero_v, deg_sh)

        pltpu.sync_copy(dst_hbm.at[pl.ds(w * rpt, rpt)], idx_v)
        plsc.subcore_barrier()

        def count_body(j, _):
            pltpu.sync_copy(ones_v.at[pl.ds(0, CH)],
                            deg_sh.at[idx_v.at[j]], add=True)
            return 0

        lax.fori_loop(0, rpt, count_body, 0)
        plsc.subcore_barrier()

        @pl.when(s == 0)
        def _():
            pltpu.sync_copy(deg_sh, deg_hbm.at[c])

    return deg_kernel


def _make_prop(n, e, d, feature_split):
    rows = e // CH
    ntiles = NS if feature_split else NC * NS
    rpt = rows // ntiles
    nbk = 16
    nblk = rpt // nbk

    @functools.partial(
        pl.kernel,
        out_type=jax.ShapeDtypeStruct((NC, n, d), jnp.float32),
        mesh=_mesh(),
        scratch_types=[
            pltpu.VMEM((nbk, CH), jnp.int32),
            pltpu.VMEM((nbk, CH), jnp.int32),
            pltpu.VMEM((nbk, CH), jnp.int32),
            pltpu.VMEM((nbk, CH), jnp.int32),
            pltpu.VMEM((CH, d), jnp.float32),
            pltpu.VMEM((CH, d), jnp.float32),
            pltpu.VMEM_SHARED((n, d), jnp.float32),
            pltpu.SemaphoreType.DMA,
            pltpu.SemaphoreType.DMA,
            pltpu.SemaphoreType.DMA,
            pltpu.SemaphoreType.DMA,
            pltpu.SemaphoreType.DMA,
            pltpu.SemaphoreType.DMA,
        ],
    )
    def prop_kernel(p_hbm, src_hbm, dst_hbm, q_hbm, src_a, dst_a, src_b,
                    dst_b, rows0_v, rows1_v, acc_sh, g0, g1, s0, s1,
                    ia, ib):
        c = lax.axis_index("c")
        s = lax.axis_index("s")
        if feature_split:
            table = p_hbm.at[c]
            base = s * rpt
        else:
            table = p_hbm
            base = (c * NS + s) * rpt

        r0 = _node_slab(s)

        @pl.when(s < NS - 1)
        def _():
            sl = pl.ds(r0, 624)
            pltpu.sync_copy(table.at[sl], acc_sh.at[sl])

        @pl.when(s == NS - 1)
        def _():
            sl = pl.ds(r0, 640)
            pltpu.sync_copy(table.at[sl], acc_sh.at[sl])

        pltpu.async_copy(src_hbm.at[pl.ds(base, nbk)], src_a, ia)
        pltpu.async_copy(dst_hbm.at[pl.ds(base, nbk)], dst_a, ia)
        plsc.subcore_barrier()

        def do_block(k, src_v, dst_v, src_o, dst_o, is_cur, is_oth):
            blk = pl.ds(base + k * nbk, nbk)
            pltpu.make_async_copy(src_hbm.at[blk], src_v, is_cur).wait()
            pltpu.make_async_copy(dst_hbm.at[blk], dst_v, is_cur).wait()

            @pl.when(k + 1 < nblk)
            def _():
                nxt = pl.ds(base + (k + 1) * nbk, nbk)
                pltpu.async_copy(src_hbm.at[nxt], src_o, is_oth)
                pltpu.async_copy(dst_hbm.at[nxt], dst_o, is_oth)

            pltpu.async_copy(table.at[src_v.at[0]], rows0_v, g0)
            pltpu.async_copy(table.at[src_v.at[1]], rows1_v, g1)

            def pair(i, _):
                pltpu.make_async_copy(table.at[src_v.at[2 * i]],
                                      rows0_v, g0).wait()
                pltpu.async_copy(rows0_v, acc_sh.at[dst_v.at[2 * i]], s0,
                                 add=True)
                pltpu.make_async_copy(table.at[src_v.at[2 * i + 1]],
                                      rows1_v, g1).wait()
                pltpu.async_copy(rows1_v, acc_sh.at[dst_v.at[2 * i + 1]], s1,
                                 add=True)
                pltpu.make_async_copy(rows0_v, acc_sh.at[dst_v.at[2 * i]],
                                      s0).wait()

                @pl.when(i < nbk // 2 - 1)
                def _():
                    pltpu.async_copy(table.at[src_v.at[2 * i + 2]],
                                     rows0_v, g0)

                pltpu.make_async_copy(rows1_v,
                                      acc_sh.at[dst_v.at[2 * i + 1]],
                                      s1).wait()

                @pl.when(i < nbk // 2 - 1)
                def _():
                    pltpu.async_copy(table.at[src_v.at[2 * i + 3]],
                                     rows1_v, g1)

                return 0

            lax.fori_loop(0, nbk // 2, pair, 0)

        def outer(k, _):
            @pl.when(k % 2 == 0)
            def _():
                do_block(k, src_a, dst_a, src_b, dst_b, ia, ib)

            @pl.when(k % 2 == 1)
            def _():
                do_block(k, src_b, dst_b, src_a, dst_a, ib, ia)

            return 0

        lax.fori_loop(0, nblk, outer, 0)
        plsc.subcore_barrier()

        @pl.when(s < NS - 1)
        def _():
            sl = pl.ds(r0, 624)
            pltpu.sync_copy(acc_sh.at[sl], q_hbm.at[c, sl])

        @pl.when(s == NS - 1)
        def _():
            sl = pl.ds(r0, 640)
            pltpu.sync_copy(acc_sh.at[sl], q_hbm.at[c, sl])

    return prop_kernel


def _dinv(degt_ref):
    return lax.rsqrt(degt_ref[:, 0:1] + degt_ref[:, 1:2] + 1.0)


def _mm1_body(degt_ref, x_ref, w1_ref, p1_ref):
    dinv = _dinv(degt_ref)
    z = jnp.dot(x_ref[...], w1_ref[...], preferred_element_type=jnp.float32)
    p = z * dinv
    h = p.shape[1] // 2
    p1_ref[0] = p[:, :h]
    p1_ref[1] = p[:, h:]


def _mm2_body(degt_ref, q1_ref, b1_ref, w2_ref, p2_ref):
    dinv = _dinv(degt_ref)
    q = jnp.concatenate([q1_ref[0], q1_ref[1]], axis=1)
    h = jnp.maximum(q * dinv + b1_ref[...], 0.0)
    z = jnp.dot(h, w2_ref[...], preferred_element_type=jnp.float32)
    p2_ref[...] = z * dinv


def _fin_body(degt_ref, q2_ref, p2_ref, b2_ref, out_ref):
    dinv = _dinv(degt_ref)
    o = (q2_ref[0] + q2_ref[1] - p2_ref[...]) * dinv + b2_ref[...]
    m = jnp.max(o, axis=1, keepdims=True)
    ssum = jnp.sum(jnp.exp(o - m), axis=1, keepdims=True)
    out_ref[...] = o - m - jnp.log(ssum)


def kernel(x, edge_index, W1, b1, W2, b2):
    n, d_in = x.shape
    d_hid = W1.shape[1]
    d_out = W2.shape[1]
    e = edge_index.shape[1]
    hh = d_hid // 2

    ei = edge_index.astype(jnp.int32)
    src2 = ei[0].reshape(e // CH, CH)
    dst2 = ei[1].reshape(e // CH, CH)

    deg = _make_deg(n, e)(dst2)
    degt = deg.T
    b1r = b1.reshape(1, d_hid)
    b2r = b2.reshape(1, d_out)

    bn = 2000
    grid = (n // bn,)
    f32 = jnp.float32

    p1 = pl.pallas_call(
        _mm1_body,
        grid=grid,
        in_specs=[
            pl.BlockSpec((bn, 2), lambda i: (i, 0)),
            pl.BlockSpec((bn, d_in), lambda i: (i, 0)),
            pl.BlockSpec((d_in, d_hid), lambda i: (0, 0)),
        ],
        out_specs=pl.BlockSpec((NC, bn, hh), lambda i: (0, i, 0)),
        out_shape=jax.ShapeDtypeStruct((NC, n, hh), f32),
    )(degt, x, W1)

    q1 = _make_prop(n, e, hh, feature_split=True)(p1, src2, dst2)

    p2 = pl.pallas_call(
        _mm2_body,
        grid=grid,
        in_specs=[
            pl.BlockSpec((bn, 2), lambda i: (i, 0)),
            pl.BlockSpec((NC, bn, hh), lambda i: (0, i, 0)),
            pl.BlockSpec((1, d_hid), lambda i: (0, 0)),
            pl.BlockSpec((d_hid, d_out), lambda i: (0, 0)),
        ],
        out_specs=pl.BlockSpec((bn, d_out), lambda i: (i, 0)),
        out_shape=jax.ShapeDtypeStruct((n, d_out), f32),
    )(degt, q1, b1r, W2)

    q2 = _make_prop(n, e, d_out, feature_split=False)(p2, src2, dst2)

    out = pl.pallas_call(
        _fin_body,
        grid=grid,
        in_specs=[
            pl.BlockSpec((bn, 2), lambda i: (i, 0)),
            pl.BlockSpec((NC, bn, d_out), lambda i: (0, i, 0)),
            pl.BlockSpec((bn, d_out), lambda i: (i, 0)),
            pl.BlockSpec((1, d_out), lambda i: (0, 0)),
        ],
        out_specs=pl.BlockSpec((bn, d_out), lambda i: (i, 0)),
        out_shape=jax.ShapeDtypeStruct((n, d_out), f32),
    )(degt, q2, p2, b2r)

    return out


# speedup vs baseline: 1.2136x; 1.2136x over previous
import functools

import jax
import jax.numpy as jnp
from jax import lax
from jax.experimental import pallas as pl
from jax.experimental.pallas import tpu as pltpu
from jax.experimental.pallas import tpu_sc as plsc

NC = 2
NS = 16
CH = 125


def _mesh():
    return plsc.VectorSubcoreMesh(core_axis_name="c", subcore_axis_name="s",
                                  num_cores=NC, num_subcores=NS)


def _node_slab(s):
    return s * 624


def _make_deg(n, e):
    rows = e // CH
    rpt = rows // (NC * NS)

    @functools.partial(
        pl.kernel,
        out_type=jax.ShapeDtypeStruct((NC, n), jnp.float32),
        mesh=_mesh(),
        scratch_types=[
            pltpu.VMEM((rpt, CH), jnp.int32),
            pltpu.VMEM((128,), jnp.float32),
            pltpu.VMEM((n,), jnp.float32),
            pltpu.VMEM_SHARED((n,), jnp.float32),
        ],
    )
    def deg_kernel(dst_hbm, deg_hbm, idx_v, ones_v, zero_v, deg_sh):
        c = lax.axis_index("c")
        s = lax.axis_index("s")
        w = c * NS + s

        for j in range(8):
            ones_v[pl.ds(j * 16, 16)] = jnp.ones((16,), jnp.float32)

        @pl.when(s == 0)
        def _():
            def zero_body(i, _):
                zero_v[pl.ds(i * 16, 16)] = jnp.zeros((16,), jnp.float32)
                return 0

            lax.fori_loop(0, n // 16, zero_body, 0)
            pltpu.sync_copy(zero_v, deg_sh)

        pltpu.sync_copy(dst_hbm.at[pl.ds(w * rpt, rpt)], idx_v)
        plsc.subcore_barrier()

        def count_body(j, _):
            pltpu.sync_copy(ones_v.at[pl.ds(0, CH)],
                            deg_sh.at[idx_v.at[j]], add=True)
            return 0

        lax.fori_loop(0, rpt, count_body, 0)
        plsc.subcore_barrier()

        @pl.when(s == 0)
        def _():
            pltpu.sync_copy(deg_sh, deg_hbm.at[c])

    return deg_kernel


def _make_prop(n, e, d, feature_split):
    rows = e // CH
    ntiles = NS if feature_split else NC * NS
    rpt = rows // ntiles
    nbk = 16
    nblk = rpt // nbk

    @functools.partial(
        pl.kernel,
        out_type=jax.ShapeDtypeStruct((NC, n, d), jnp.float32),
        mesh=_mesh(),
        scratch_types=[
            pltpu.VMEM((nbk, CH), jnp.int32),
            pltpu.VMEM((nbk, CH), jnp.int32),
            pltpu.VMEM((nbk, CH), jnp.int32),
            pltpu.VMEM((nbk, CH), jnp.int32),
            pltpu.VMEM((CH, d), jnp.float32),
            pltpu.VMEM((CH, d), jnp.float32),
            pltpu.VMEM_SHARED((n, d), jnp.float32),
            pltpu.SemaphoreType.DMA,
            pltpu.SemaphoreType.DMA,
            pltpu.SemaphoreType.DMA,
            pltpu.SemaphoreType.DMA,
            pltpu.SemaphoreType.DMA,
            pltpu.SemaphoreType.DMA,
        ],
    )
    def prop_kernel(p_hbm, src_hbm, dst_hbm, q_hbm, src_a, dst_a, src_b,
                    dst_b, rows0_v, rows1_v, acc_sh, g0, g1, s0, s1,
                    ia, ib):
        c = lax.axis_index("c")
        s = lax.axis_index("s")
        if feature_split:
            table = p_hbm.at[c]
            base = s * rpt
        else:
            table = p_hbm
            base = (c * NS + s) * rpt

        r0 = _node_slab(s)

        @pl.when(s < NS - 1)
        def _():
            sl = pl.ds(r0, 624)
            pltpu.sync_copy(table.at[sl], acc_sh.at[sl])

        @pl.when(s == NS - 1)
        def _():
            sl = pl.ds(r0, 640)
            pltpu.sync_copy(table.at[sl], acc_sh.at[sl])

        pltpu.async_copy(src_hbm.at[pl.ds(base, nbk)], src_a, ia)
        pltpu.async_copy(dst_hbm.at[pl.ds(base, nbk)], dst_a, ia)
        plsc.subcore_barrier()

        def do_block(k, src_v, dst_v, src_o, dst_o, is_cur, is_oth):
            blk = pl.ds(base + k * nbk, nbk)
            pltpu.make_async_copy(src_hbm.at[blk], src_v, is_cur).wait()
            pltpu.make_async_copy(dst_hbm.at[blk], dst_v, is_cur).wait()

            @pl.when(k + 1 < nblk)
            def _():
                nxt = pl.ds(base + (k + 1) * nbk, nbk)
                pltpu.async_copy(src_hbm.at[nxt], src_o, is_oth)
                pltpu.async_copy(dst_hbm.at[nxt], dst_o, is_oth)

            pltpu.async_copy(table.at[src_v.at[0]], rows0_v, g0)
            pltpu.async_copy(table.at[src_v.at[1]], rows1_v, g1)

            def pair(i, _):
                pltpu.make_async_copy(table.at[src_v.at[2 * i]],
                                      rows0_v, g0).wait()
                pltpu.sync_copy(rows0_v, acc_sh.at[dst_v.at[2 * i]], add=True)
                pltpu.make_async_copy(table.at[src_v.at[2 * i + 1]],
                                      rows1_v, g1).wait()

                @pl.when(i < nbk // 2 - 1)
                def _():
                    pltpu.async_copy(table.at[src_v.at[2 * i + 2]],
                                     rows0_v, g0)

                pltpu.sync_copy(rows1_v, acc_sh.at[dst_v.at[2 * i + 1]],
                                add=True)

                @pl.when(i < nbk // 2 - 1)
                def _():
                    pltpu.async_copy(table.at[src_v.at[2 * i + 3]],
                                     rows1_v, g1)

                return 0

            lax.fori_loop(0, nbk // 2, pair, 0)

        def outer(k, _):
            @pl.when(k % 2 == 0)
            def _():
                do_block(k, src_a, dst_a, src_b, dst_b, ia, ib)

            @pl.when(k % 2 == 1)
            def _():
                do_block(k, src_b, dst_b, src_a, dst_a, ib, ia)

            return 0

        lax.fori_loop(0, nblk, outer, 0)
        plsc.subcore_barrier()

        @pl.when(s < NS - 1)
        def _():
            sl = pl.ds(r0, 624)
            pltpu.sync_copy(acc_sh.at[sl], q_hbm.at[c, sl])

        @pl.when(s == NS - 1)
        def _():
            sl = pl.ds(r0, 640)
            pltpu.sync_copy(acc_sh.at[sl], q_hbm.at[c, sl])

    return prop_kernel


def _dinv(degt_ref):
    return lax.rsqrt(degt_ref[:, 0:1] + degt_ref[:, 1:2] + 1.0)


def _mm1_body(degt_ref, x_ref, w1_ref, p1_ref):
    dinv = _dinv(degt_ref)
    z = jnp.dot(x_ref[...], w1_ref[...], preferred_element_type=jnp.float32)
    p = z * dinv
    h = p.shape[1] // 2
    p1_ref[0] = p[:, :h]
    p1_ref[1] = p[:, h:]


def _mm2_body(degt_ref, q1_ref, b1_ref, w2_ref, p2_ref):
    dinv = _dinv(degt_ref)
    q = jnp.concatenate([q1_ref[0], q1_ref[1]], axis=1)
    h = jnp.maximum(q * dinv + b1_ref[...], 0.0)
    z = jnp.dot(h, w2_ref[...], preferred_element_type=jnp.float32)
    p2_ref[...] = z * dinv


def _fin_body(degt_ref, q2_ref, p2_ref, b2_ref, out_ref):
    dinv = _dinv(degt_ref)
    o = (q2_ref[0] + q2_ref[1] - p2_ref[...]) * dinv + b2_ref[...]
    m = jnp.max(o, axis=1, keepdims=True)
    ssum = jnp.sum(jnp.exp(o - m), axis=1, keepdims=True)
    out_ref[...] = o - m - jnp.log(ssum)


def kernel(x, edge_index, W1, b1, W2, b2):
    n, d_in = x.shape
    d_hid = W1.shape[1]
    d_out = W2.shape[1]
    e = edge_index.shape[1]
    hh = d_hid // 2

    ei = edge_index.astype(jnp.int32)
    src2 = ei[0].reshape(e // CH, CH)
    dst2 = ei[1].reshape(e // CH, CH)

    deg = _make_deg(n, e)(dst2)
    degt = deg.T
    b1r = b1.reshape(1, d_hid)
    b2r = b2.reshape(1, d_out)

    bn = 2000
    grid = (n // bn,)
    f32 = jnp.float32

    p1 = pl.pallas_call(
        _mm1_body,
        grid=grid,
        in_specs=[
            pl.BlockSpec((bn, 2), lambda i: (i, 0)),
            pl.BlockSpec((bn, d_in), lambda i: (i, 0)),
            pl.BlockSpec((d_in, d_hid), lambda i: (0, 0)),
        ],
        out_specs=pl.BlockSpec((NC, bn, hh), lambda i: (0, i, 0)),
        out_shape=jax.ShapeDtypeStruct((NC, n, hh), f32),
    )(degt, x, W1)

    q1 = _make_prop(n, e, hh, feature_split=True)(p1, src2, dst2)

    p2 = pl.pallas_call(
        _mm2_body,
        grid=grid,
        in_specs=[
            pl.BlockSpec((bn, 2), lambda i: (i, 0)),
            pl.BlockSpec((NC, bn, hh), lambda i: (0, i, 0)),
            pl.BlockSpec((1, d_hid), lambda i: (0, 0)),
            pl.BlockSpec((d_hid, d_out), lambda i: (0, 0)),
        ],
        out_specs=pl.BlockSpec((bn, d_out), lambda i: (i, 0)),
        out_shape=jax.ShapeDtypeStruct((n, d_out), f32),
    )(degt, q1, b1r, W2)

    q2 = _make_prop(n, e, d_out, feature_split=False)(p2, src2, dst2)

    out = pl.pallas_call(
        _fin_body,
        grid=grid,
        in_specs=[
            pl.BlockSpec((bn, 2), lambda i: (i, 0)),
            pl.BlockSpec((NC, bn, d_out), lambda i: (0, i, 0)),
            pl.BlockSpec((bn, d_out), lambda i: (i, 0)),
            pl.BlockSpec((1, d_out), lambda i: (0, 0)),
        ],
        out_specs=pl.BlockSpec((bn, d_out), lambda i: (i, 0)),
        out_shape=jax.ShapeDtypeStruct((n, d_out), f32),
    )(degt, q2, p2, b2r)

    return out


# speedup vs baseline: 1.2515x; 1.0312x over previous
import functools

import jax
import jax.numpy as jnp
from jax import lax
from jax.experimental import pallas as pl
from jax.experimental.pallas import tpu as pltpu
from jax.experimental.pallas import tpu_sc as plsc

NC = 2
NS = 16
CH = 125


def _mesh():
    return plsc.VectorSubcoreMesh(core_axis_name="c", subcore_axis_name="s",
                                  num_cores=NC, num_subcores=NS)


def _node_slab(s):
    return s * 624


def _make_deg(n, e):
    rows = e // CH
    rpt = rows // (NC * NS)

    @functools.partial(
        pl.kernel,
        out_type=jax.ShapeDtypeStruct((NC, n), jnp.float32),
        mesh=_mesh(),
        scratch_types=[
            pltpu.VMEM((rpt, CH), jnp.int32),
            pltpu.VMEM((128,), jnp.float32),
            pltpu.VMEM((n,), jnp.float32),
            pltpu.VMEM_SHARED((n,), jnp.float32),
        ],
    )
    def deg_kernel(ei_hbm, deg_hbm, idx_v, ones_v, zero_v, deg_sh):
        dst_hbm = ei_hbm.at[1]
        c = lax.axis_index("c")
        s = lax.axis_index("s")
        w = c * NS + s

        for j in range(8):
            ones_v[pl.ds(j * 16, 16)] = jnp.ones((16,), jnp.float32)

        @pl.when(s == 0)
        def _():
            def zero_body(i, _):
                zero_v[pl.ds(i * 16, 16)] = jnp.zeros((16,), jnp.float32)
                return 0

            lax.fori_loop(0, n // 16, zero_body, 0)
            pltpu.sync_copy(zero_v, deg_sh)

        pltpu.sync_copy(dst_hbm.at[pl.ds(w * rpt, rpt)], idx_v)
        plsc.subcore_barrier()

        def count_body(j, _):
            pltpu.sync_copy(ones_v.at[pl.ds(0, CH)],
                            deg_sh.at[idx_v.at[j]], add=True)
            return 0

        lax.fori_loop(0, rpt, count_body, 0)
        plsc.subcore_barrier()

        @pl.when(s == 0)
        def _():
            pltpu.sync_copy(deg_sh, deg_hbm.at[c])

    return deg_kernel


def _make_prop(n, e, d, feature_split):
    rows = e // CH
    ntiles = NS if feature_split else NC * NS
    rpt = rows // ntiles
    nbk = 16
    nblk = rpt // nbk

    @functools.partial(
        pl.kernel,
        out_type=jax.ShapeDtypeStruct((NC, n, d), jnp.float32),
        mesh=_mesh(),
        scratch_types=[
            pltpu.VMEM((nbk, CH), jnp.int32),
            pltpu.VMEM((nbk, CH), jnp.int32),
            pltpu.VMEM((nbk, CH), jnp.int32),
            pltpu.VMEM((nbk, CH), jnp.int32),
            pltpu.VMEM((CH, d), jnp.float32),
            pltpu.VMEM((CH, d), jnp.float32),
            pltpu.VMEM_SHARED((n, d), jnp.float32),
            pltpu.SemaphoreType.DMA,
            pltpu.SemaphoreType.DMA,
            pltpu.SemaphoreType.DMA,
            pltpu.SemaphoreType.DMA,
            pltpu.SemaphoreType.DMA,
            pltpu.SemaphoreType.DMA,
        ],
    )
    def prop_kernel(p_hbm, ei_hbm, q_hbm, src_a, dst_a, src_b,
                    dst_b, rows0_v, rows1_v, acc_sh, g0, g1, s0, s1,
                    ia, ib):
        src_hbm = ei_hbm.at[0]
        dst_hbm = ei_hbm.at[1]
        c = lax.axis_index("c")
        s = lax.axis_index("s")
        if feature_split:
            table = p_hbm.at[c]
            base = s * rpt
        else:
            table = p_hbm
            base = (c * NS + s) * rpt

        pltpu.async_copy(src_hbm.at[pl.ds(base, nbk)], src_a, ia)
        pltpu.async_copy(dst_hbm.at[pl.ds(base, nbk)], dst_a, ia)

        r0 = _node_slab(s)

        @pl.when(s < NS - 1)
        def _():
            sl = pl.ds(r0, 624)
            pltpu.sync_copy(table.at[sl], acc_sh.at[sl])

        @pl.when(s == NS - 1)
        def _():
            sl = pl.ds(r0, 640)
            pltpu.sync_copy(table.at[sl], acc_sh.at[sl])

        pltpu.make_async_copy(src_hbm.at[pl.ds(base, nbk)], src_a, ia).wait()
        pltpu.make_async_copy(dst_hbm.at[pl.ds(base, nbk)], dst_a, ia).wait()
        pltpu.async_copy(table.at[src_a.at[0]], rows0_v, g0)
        pltpu.async_copy(table.at[src_a.at[1]], rows1_v, g1)
        plsc.subcore_barrier()

        def do_block(k, src_v, dst_v, src_o, dst_o, is_oth):
            @pl.when(k + 1 < nblk)
            def _():
                nxt = pl.ds(base + (k + 1) * nbk, nbk)
                pltpu.async_copy(src_hbm.at[nxt], src_o, is_oth)
                pltpu.async_copy(dst_hbm.at[nxt], dst_o, is_oth)

            def pair(i, _):
                pltpu.make_async_copy(table.at[src_v.at[2 * i]],
                                      rows0_v, g0).wait()
                pltpu.sync_copy(rows0_v, acc_sh.at[dst_v.at[2 * i]], add=True)
                pltpu.make_async_copy(table.at[src_v.at[2 * i + 1]],
                                      rows1_v, g1).wait()

                @pl.when(i < nbk // 2 - 1)
                def _():
                    pltpu.async_copy(table.at[src_v.at[2 * i + 2]],
                                     rows0_v, g0)

                pltpu.sync_copy(rows1_v, acc_sh.at[dst_v.at[2 * i + 1]],
                                add=True)

                @pl.when(i < nbk // 2 - 1)
                def _():
                    pltpu.async_copy(table.at[src_v.at[2 * i + 3]],
                                     rows1_v, g1)

                @pl.when((i == nbk // 2 - 1) & (k + 1 < nblk))
                def _():
                    nxt = pl.ds(base + (k + 1) * nbk, nbk)
                    pltpu.make_async_copy(src_hbm.at[nxt], src_o,
                                          is_oth).wait()
                    pltpu.make_async_copy(dst_hbm.at[nxt], dst_o,
                                          is_oth).wait()
                    pltpu.async_copy(table.at[src_o.at[0]], rows0_v, g0)
                    pltpu.async_copy(table.at[src_o.at[1]], rows1_v, g1)

                return 0

            lax.fori_loop(0, nbk // 2, pair, 0)

        def outer(k, _):
            @pl.when(k % 2 == 0)
            def _():
                do_block(k, src_a, dst_a, src_b, dst_b, ib)

            @pl.when(k % 2 == 1)
            def _():
                do_block(k, src_b, dst_b, src_a, dst_a, ia)

            return 0

        lax.fori_loop(0, nblk, outer, 0)
        plsc.subcore_barrier()

        @pl.when(s < NS - 1)
        def _():
            sl = pl.ds(r0, 624)
            pltpu.sync_copy(acc_sh.at[sl], q_hbm.at[c, sl])

        @pl.when(s == NS - 1)
        def _():
            sl = pl.ds(r0, 640)
            pltpu.sync_copy(acc_sh.at[sl], q_hbm.at[c, sl])

    return prop_kernel


def _dinv(degt_ref):
    return lax.rsqrt(degt_ref[:, 0:1] + degt_ref[:, 1:2] + 1.0)


def _mm1_body(degt_ref, x_ref, w1_ref, p1_ref):
    dinv = _dinv(degt_ref)
    z = jnp.dot(x_ref[...], w1_ref[...], preferred_element_type=jnp.float32)
    p = z * dinv
    h = p.shape[1] // 2
    p1_ref[0] = p[:, :h]
    p1_ref[1] = p[:, h:]


def _mm2_body(degt_ref, q1_ref, b1_ref, w2_ref, p2_ref):
    dinv = _dinv(degt_ref)
    q = jnp.concatenate([q1_ref[0], q1_ref[1]], axis=1)
    h = jnp.maximum(q * dinv + b1_ref[...], 0.0)
    z = jnp.dot(h, w2_ref[...], preferred_element_type=jnp.float32)
    p2_ref[...] = z * dinv


def _fin_body(degt_ref, q2_ref, p2_ref, b2_ref, out_ref):
    dinv = _dinv(degt_ref)
    o = (q2_ref[0] + q2_ref[1] - p2_ref[...]) * dinv + b2_ref[...]
    m = jnp.max(o, axis=1, keepdims=True)
    ssum = jnp.sum(jnp.exp(o - m), axis=1, keepdims=True)
    out_ref[...] = o - m - jnp.log(ssum)


def kernel(x, edge_index, W1, b1, W2, b2):
    n, d_in = x.shape
    d_hid = W1.shape[1]
    d_out = W2.shape[1]
    e = edge_index.shape[1]
    hh = d_hid // 2

    ei2 = edge_index.astype(jnp.int32).reshape(2, e // CH, CH)

    deg = _make_deg(n, e)(ei2)
    degt = deg.T
    b1r = b1.reshape(1, d_hid)
    b2r = b2.reshape(1, d_out)

    bn = 2000
    grid = (n // bn,)
    f32 = jnp.float32

    p1 = pl.pallas_call(
        _mm1_body,
        grid=grid,
        in_specs=[
            pl.BlockSpec((bn, 2), lambda i: (i, 0)),
            pl.BlockSpec((bn, d_in), lambda i: (i, 0)),
            pl.BlockSpec((d_in, d_hid), lambda i: (0, 0)),
        ],
        out_specs=pl.BlockSpec((NC, bn, hh), lambda i: (0, i, 0)),
        out_shape=jax.ShapeDtypeStruct((NC, n, hh), f32),
    )(degt, x, W1)

    q1 = _make_prop(n, e, hh, feature_split=True)(p1, ei2)

    p2 = pl.pallas_call(
        _mm2_body,
        grid=grid,
        in_specs=[
            pl.BlockSpec((bn, 2), lambda i: (i, 0)),
            pl.BlockSpec((NC, bn, hh), lambda i: (0, i, 0)),
            pl.BlockSpec((1, d_hid), lambda i: (0, 0)),
            pl.BlockSpec((d_hid, d_out), lambda i: (0, 0)),
        ],
        out_specs=pl.BlockSpec((bn, d_out), lambda i: (i, 0)),
        out_shape=jax.ShapeDtypeStruct((n, d_out), f32),
    )(degt, q1, b1r, W2)

    q2 = _make_prop(n, e, d_out, feature_split=False)(p2, ei2)

    out = pl.pallas_call(
        _fin_body,
        grid=grid,
        in_specs=[
            pl.BlockSpec((bn, 2), lambda i: (i, 0)),
            pl.BlockSpec((NC, bn, d_out), lambda i: (0, i, 0)),
            pl.BlockSpec((bn, d_out), lambda i: (i, 0)),
            pl.BlockSpec((1, d_out), lambda i: (0, 0)),
        ],
        out_specs=pl.BlockSpec((bn, d_out), lambda i: (i, 0)),
        out_shape=jax.ShapeDtypeStruct((n, d_out), f32),
    )(degt, q2, p2, b2r)

    return out


# speedup vs baseline: 1.2753x; 1.0191x over previous
import functools

import jax
import jax.numpy as jnp
from jax import lax
from jax.experimental import pallas as pl
from jax.experimental.pallas import tpu as pltpu
from jax.experimental.pallas import tpu_sc as plsc

NC = 2
NS = 16
CH = 125


def _mesh():
    return plsc.VectorSubcoreMesh(core_axis_name="c", subcore_axis_name="s",
                                  num_cores=NC, num_subcores=NS)


def _node_slab(s):
    return s * 624


def _make_deg(n, e):
    rows = e // CH
    rpt = rows // (NC * NS)

    @functools.partial(
        pl.kernel,
        out_type=jax.ShapeDtypeStruct((NC, n), jnp.float32),
        mesh=_mesh(),
        scratch_types=[
            pltpu.VMEM((rpt, CH), jnp.int32),
            pltpu.VMEM((128,), jnp.float32),
            pltpu.VMEM((640,), jnp.float32),
            pltpu.VMEM_SHARED((n,), jnp.float32),
            pltpu.SemaphoreType.DMA,
            pltpu.SemaphoreType.DMA,
        ],
    )
    def deg_kernel(ei_hbm, deg_hbm, idx_v, ones_v, zero_v, deg_sh, isem,
                   ssem):
        dst_hbm = ei_hbm.at[1]
        c = lax.axis_index("c")
        s = lax.axis_index("s")
        w = c * NS + s

        pltpu.async_copy(dst_hbm.at[pl.ds(w * rpt, rpt)], idx_v, isem)

        for j in range(8):
            ones_v[pl.ds(j * 16, 16)] = jnp.ones((16,), jnp.float32)

        def zero_body(i, _):
            zero_v[pl.ds(i * 16, 16)] = jnp.zeros((16,), jnp.float32)
            return 0

        lax.fori_loop(0, 640 // 16, zero_body, 0)
        r0 = _node_slab(s)

        @pl.when(s < NS - 1)
        def _():
            pltpu.sync_copy(zero_v.at[pl.ds(0, 624)],
                            deg_sh.at[pl.ds(r0, 624)])

        @pl.when(s == NS - 1)
        def _():
            pltpu.sync_copy(zero_v, deg_sh.at[pl.ds(r0, 640)])

        pltpu.make_async_copy(dst_hbm.at[pl.ds(w * rpt, rpt)], idx_v,
                              isem).wait()
        plsc.subcore_barrier()

        ones = ones_v.at[pl.ds(0, CH)]

        def count_body(j, _):
            for u in range(8):
                pltpu.async_copy(ones, deg_sh.at[idx_v.at[8 * j + u]], ssem,
                                 add=True)
            for u in range(8):
                pltpu.make_async_copy(ones, deg_sh.at[idx_v.at[8 * j + u]],
                                      ssem).wait()
            return 0

        lax.fori_loop(0, rpt // 8, count_body, 0)
        plsc.subcore_barrier()

        @pl.when(s == 0)
        def _():
            pltpu.sync_copy(deg_sh, deg_hbm.at[c])

    return deg_kernel


def _make_prop(n, e, d, feature_split):
    rows = e // CH
    ntiles = NS if feature_split else NC * NS
    rpt = rows // ntiles
    nbk = 16
    nblk = rpt // nbk

    @functools.partial(
        pl.kernel,
        out_type=jax.ShapeDtypeStruct((NC, n, d), jnp.float32),
        mesh=_mesh(),
        scratch_types=[
            pltpu.VMEM((nbk, CH), jnp.int32),
            pltpu.VMEM((nbk, CH), jnp.int32),
            pltpu.VMEM((nbk, CH), jnp.int32),
            pltpu.VMEM((nbk, CH), jnp.int32),
            pltpu.VMEM((CH, d), jnp.float32),
            pltpu.VMEM((CH, d), jnp.float32),
            pltpu.VMEM_SHARED((n, d), jnp.float32),
            pltpu.SemaphoreType.DMA,
            pltpu.SemaphoreType.DMA,
            pltpu.SemaphoreType.DMA,
            pltpu.SemaphoreType.DMA,
            pltpu.SemaphoreType.DMA,
            pltpu.SemaphoreType.DMA,
        ],
    )
    def prop_kernel(p_hbm, ei_hbm, q_hbm, src_a, dst_a, src_b,
                    dst_b, rows0_v, rows1_v, acc_sh, g0, g1, s0, s1,
                    ia, ib):
        src_hbm = ei_hbm.at[0]
        dst_hbm = ei_hbm.at[1]
        c = lax.axis_index("c")
        s = lax.axis_index("s")
        if feature_split:
            table = p_hbm.at[c]
            base = s * rpt
        else:
            table = p_hbm
            base = (c * NS + s) * rpt

        pltpu.async_copy(src_hbm.at[pl.ds(base, nbk)], src_a, ia)
        pltpu.async_copy(dst_hbm.at[pl.ds(base, nbk)], dst_a, ia)

        r0 = _node_slab(s)

        @pl.when(s < NS - 1)
        def _():
            sl = pl.ds(r0, 624)
            pltpu.sync_copy(table.at[sl], acc_sh.at[sl])

        @pl.when(s == NS - 1)
        def _():
            sl = pl.ds(r0, 640)
            pltpu.sync_copy(table.at[sl], acc_sh.at[sl])

        pltpu.make_async_copy(src_hbm.at[pl.ds(base, nbk)], src_a, ia).wait()
        pltpu.make_async_copy(dst_hbm.at[pl.ds(base, nbk)], dst_a, ia).wait()
        pltpu.async_copy(table.at[src_a.at[0]], rows0_v, g0)
        pltpu.async_copy(table.at[src_a.at[1]], rows1_v, g1)
        plsc.subcore_barrier()

        def do_block(k, src_v, dst_v, src_o, dst_o, is_oth):
            @pl.when(k + 1 < nblk)
            def _():
                nxt = pl.ds(base + (k + 1) * nbk, nbk)
                pltpu.async_copy(src_hbm.at[nxt], src_o, is_oth)
                pltpu.async_copy(dst_hbm.at[nxt], dst_o, is_oth)

            def pair(i, _):
                pltpu.make_async_copy(table.at[src_v.at[2 * i]],
                                      rows0_v, g0).wait()
                pltpu.sync_copy(rows0_v, acc_sh.at[dst_v.at[2 * i]], add=True)
                pltpu.make_async_copy(table.at[src_v.at[2 * i + 1]],
                                      rows1_v, g1).wait()

                @pl.when(i < nbk // 2 - 1)
                def _():
                    pltpu.async_copy(table.at[src_v.at[2 * i + 2]],
                                     rows0_v, g0)

                pltpu.sync_copy(rows1_v, acc_sh.at[dst_v.at[2 * i + 1]],
                                add=True)

                @pl.when(i < nbk // 2 - 1)
                def _():
                    pltpu.async_copy(table.at[src_v.at[2 * i + 3]],
                                     rows1_v, g1)

                @pl.when((i == nbk // 2 - 1) & (k + 1 < nblk))
                def _():
                    nxt = pl.ds(base + (k + 1) * nbk, nbk)
                    pltpu.make_async_copy(src_hbm.at[nxt], src_o,
                                          is_oth).wait()
                    pltpu.make_async_copy(dst_hbm.at[nxt], dst_o,
                                          is_oth).wait()
                    pltpu.async_copy(table.at[src_o.at[0]], rows0_v, g0)
                    pltpu.async_copy(table.at[src_o.at[1]], rows1_v, g1)

                return 0

            lax.fori_loop(0, nbk // 2, pair, 0)

        def outer(k, _):
            @pl.when(k % 2 == 0)
            def _():
                do_block(k, src_a, dst_a, src_b, dst_b, ib)

            @pl.when(k % 2 == 1)
            def _():
                do_block(k, src_b, dst_b, src_a, dst_a, ia)

            return 0

        lax.fori_loop(0, nblk, outer, 0)
        plsc.subcore_barrier()

        @pl.when(s < NS - 1)
        def _():
            sl = pl.ds(r0, 624)
            pltpu.sync_copy(acc_sh.at[sl], q_hbm.at[c, sl])

        @pl.when(s == NS - 1)
        def _():
            sl = pl.ds(r0, 640)
            pltpu.sync_copy(acc_sh.at[sl], q_hbm.at[c, sl])

    return prop_kernel


def _dinv(degt_ref):
    return lax.rsqrt(degt_ref[:, 0:1] + degt_ref[:, 1:2] + 1.0)


def _mm1_body(degt_ref, x_ref, w1_ref, p1_ref):
    dinv = _dinv(degt_ref)
    z = jnp.dot(x_ref[...], w1_ref[...], preferred_element_type=jnp.float32)
    p = z * dinv
    h = p.shape[1] // 2
    p1_ref[0] = p[:, :h]
    p1_ref[1] = p[:, h:]


def _mm2_body(degt_ref, q1_ref, b1_ref, w2_ref, p2_ref):
    dinv = _dinv(degt_ref)
    q = jnp.concatenate([q1_ref[0], q1_ref[1]], axis=1)
    h = jnp.maximum(q * dinv + b1_ref[...], 0.0)
    z = jnp.dot(h, w2_ref[...], preferred_element_type=jnp.float32)
    p2_ref[...] = z * dinv


def _fin_body(degt_ref, q2_ref, p2_ref, b2_ref, out_ref):
    dinv = _dinv(degt_ref)
    o = (q2_ref[0] + q2_ref[1] - p2_ref[...]) * dinv + b2_ref[...]
    m = jnp.max(o, axis=1, keepdims=True)
    ssum = jnp.sum(jnp.exp(o - m), axis=1, keepdims=True)
    out_ref[...] = o - m - jnp.log(ssum)


def kernel(x, edge_index, W1, b1, W2, b2):
    n, d_in = x.shape
    d_hid = W1.shape[1]
    d_out = W2.shape[1]
    e = edge_index.shape[1]
    hh = d_hid // 2

    ei2 = edge_index.astype(jnp.int32).reshape(2, e // CH, CH)

    deg = _make_deg(n, e)(ei2)
    degt = deg.T
    b1r = b1.reshape(1, d_hid)
    b2r = b2.reshape(1, d_out)

    bn = 2000
    grid = (n // bn,)
    f32 = jnp.float32

    p1 = pl.pallas_call(
        _mm1_body,
        grid=grid,
        in_specs=[
            pl.BlockSpec((bn, 2), lambda i: (i, 0)),
            pl.BlockSpec((bn, d_in), lambda i: (i, 0)),
            pl.BlockSpec((d_in, d_hid), lambda i: (0, 0)),
        ],
        out_specs=pl.BlockSpec((NC, bn, hh), lambda i: (0, i, 0)),
        out_shape=jax.ShapeDtypeStruct((NC, n, hh), f32),
    )(degt, x, W1)

    q1 = _make_prop(n, e, hh, feature_split=True)(p1, ei2)

    p2 = pl.pallas_call(
        _mm2_body,
        grid=grid,
        in_specs=[
            pl.BlockSpec((bn, 2), lambda i: (i, 0)),
            pl.BlockSpec((NC, bn, hh), lambda i: (0, i, 0)),
            pl.BlockSpec((1, d_hid), lambda i: (0, 0)),
            pl.BlockSpec((d_hid, d_out), lambda i: (0, 0)),
        ],
        out_specs=pl.BlockSpec((bn, d_out), lambda i: (i, 0)),
        out_shape=jax.ShapeDtypeStruct((n, d_out), f32),
    )(degt, q1, b1r, W2)

    q2 = _make_prop(n, e, d_out, feature_split=False)(p2, ei2)

    out = pl.pallas_call(
        _fin_body,
        grid=grid,
        in_specs=[
            pl.BlockSpec((bn, 2), lambda i: (i, 0)),
            pl.BlockSpec((NC, bn, d_out), lambda i: (0, i, 0)),
            pl.BlockSpec((bn, d_out), lambda i: (i, 0)),
            pl.BlockSpec((1, d_out), lambda i: (0, 0)),
        ],
        out_specs=pl.BlockSpec((bn, d_out), lambda i: (i, 0)),
        out_shape=jax.ShapeDtypeStruct((n, d_out), f32),
    )(degt, q2, p2, b2r)

    return out
